# baseline (speedup 1.0000x reference)
import functools

import jax
import jax.numpy as jnp
from jax import lax
from jax.experimental import pallas as pl
from jax.experimental.pallas import tpu as pltpu
from jax.experimental.pallas import tpu_sc as plsc

NC = 2
NS = 16
NW = NC * NS
L = 16


def _build(M, K, NF, NP):
    rpw = M // NW
    CW = 128
    C = rpw // CW
    P = 32
    NPIECE = CW // P
    TP = C * NPIECE
    NG = NP // L
    PW = P * K
    JT = (NF + 7) // 8
    IT = M // 128

    mesh = plsc.VectorSubcoreMesh(core_axis_name="c", subcore_axis_name="s")

    @functools.partial(
        pl.kernel,
        out_type=jax.ShapeDtypeStruct((JT, IT * 1024), jnp.float32),
        mesh=mesh,
        scratch_types=[
            pltpu.VMEM((NP,), jnp.int32),
            pltpu.VMEM((NP,), jnp.int32),
            pltpu.VMEM((NP,), jnp.int32),
            pltpu.VMEM((NP,), jnp.int32),
            pltpu.VMEM((PW,), jnp.float32),
            pltpu.VMEM((PW,), jnp.float32),
            pltpu.VMEM((JT, 1024), jnp.float32),
            pltpu.VMEM((JT, 1024), jnp.float32),
            pltpu.SemaphoreType.DMA,
            pltpu.SemaphoreType.DMA,
            pltpu.SemaphoreType.DMA,
            pltpu.SemaphoreType.DMA,
        ],
        compiler_params=pltpu.CompilerParams(
            use_tc_tiling_on_sc=False,
            needs_layout_passes=False,
            disable_bounds_checks=True,
        ),
    )
    def k(x_hbm, idx_hbm, out_hbm, idxv, colt, rvt, cbt, xpa, xpb,
          outa, outb, isa, isb, osa, osb):
        xps, outs = [xpa, xpb], [outa, outb]
        isems, osems = [isa, isb], [osa, osb]
        wid = lax.axis_index("s") * NC + lax.axis_index("c")
        row0 = wid * rpw
        iota = lax.iota(jnp.int32, L)

        def issue_in(n, b):
            return pltpu.async_copy(
                x_hbm.at[pl.ds((row0 + n * P) * K, PW)], xps[b], isems[b]
            )

        issue_in(0, 0)

        pltpu.sync_copy(idx_hbm, idxv)
        for g in range(NG):
            v = idxv[pl.ds(g * L, L)]
            colt[pl.ds(g * L, L)] = (v >> 7) * 1024 + (v & 127)
            j0 = g * L if g < NG - 1 else NF - L
            jv = iota + j0
            rvt[pl.ds(g * L, L)] = jv >> 3
            cbt[pl.ds(g * L, L)] = (jv & 7) * 128

        def wait_in(b):
            pltpu.make_async_copy(
                x_hbm.at[pl.ds(0, PW)], xps[b], isems[b]
            ).wait()

        def issue_out(ch, b):
            return pltpu.async_copy(
                outs[b],
                out_hbm.at[:, pl.ds((wid * C + ch) * 1024, 1024)],
                osems[b],
            )

        def wait_out(b):
            pltpu.make_async_copy(
                outs[b], out_hbm.at[:, pl.ds(0, 1024)], osems[b]
            ).wait()

        def compute_piece(xp, outv, col0):
            @plsc.parallel_loop(0, NG)
            def gbody(g):
                g16 = pl.multiple_of(g * L, L)
                colp = colt[pl.ds(g16, L)]
                rv = rvt[pl.ds(g16, L)]
                cb = cbt[pl.ds(g16, L)] + col0

                @plsc.parallel_loop(0, L)
                def tbody(t):
                    m = (iota + t) & 15
                    ro = ((m >> 3) << 12) + ((m & 7) << 7)
                    gb = colp + ro
                    sb = cb + m
                    for b in range(P // L):
                        vals = plsc.load_gather(xp, [gb + b * 8192])
                        plsc.store_scatter(outv, [rv, sb + b * L], vals)

        def citer(it, _):
            for cc in range(2):
                ch = 2 * it + cc
                for q in range(NPIECE):
                    n = ch * NPIECE + q
                    wait_in(q & 1)

                    @pl.when(n + 1 < TP)
                    def _():
                        issue_in(n + 1, (q + 1) & 1)

                    if q == 0:
                        @pl.when(ch >= 2)
                        def _():
                            wait_out(cc)

                    compute_piece(xps[q & 1], outs[cc], q * P)
                issue_out(ch, cc)
            return 0

        lax.fori_loop(0, C // 2, citer, 0)
        wait_out(0)
        wait_out(1)

    return k


def kernel(x, feature_indices):
    M, K = x.shape
    NF = feature_indices.shape[0]
    G = NF // L
    rem = NF % L
    if rem:
        idx_pad = jnp.concatenate(
            [feature_indices[: G * L], feature_indices[NF - L :]]
        )
    else:
        idx_pad = feature_indices
    NP = idx_pad.shape[0]

    x1 = jnp.transpose(
        x.reshape(M // 8, 8, K // 128, 128), (0, 2, 1, 3)
    ).reshape(-1)

    k = _build(M, K, NF, NP)
    out1 = k(x1, idx_pad.astype(jnp.int32))

    JT = (NF + 7) // 8
    outT = jnp.transpose(
        out1.reshape(JT, M // 128, 8, 128), (0, 2, 1, 3)
    ).reshape(JT * 8, M)
    return outT[:NF].T


# speedup vs baseline: 1.3416x; 1.3416x over previous
import functools

import jax
import jax.numpy as jnp
from jax import lax
from jax.experimental import pallas as pl
from jax.experimental.pallas import tpu as pltpu
from jax.experimental.pallas import tpu_sc as plsc

NC = 2
NS = 16
NW = NC * NS
L = 16


def _build(M, K, NF, NP):
    rpw = M // NW
    CW = 128
    C = rpw // CW
    P = 32
    NPIECE = CW // P
    TP = C * NPIECE
    NG = NP // L
    PW = P * K

    mesh = plsc.VectorSubcoreMesh(core_axis_name="c", subcore_axis_name="s")

    @functools.partial(
        pl.kernel,
        out_type=jax.ShapeDtypeStruct((NF, M), jnp.float32),
        mesh=mesh,
        scratch_types=[
            pltpu.VMEM((NP,), jnp.int32),
            pltpu.VMEM((NP,), jnp.int32),
            pltpu.VMEM((NP,), jnp.int32),
            pltpu.VMEM((PW,), jnp.float32),
            pltpu.VMEM((PW,), jnp.float32),
            pltpu.VMEM((NF, CW), jnp.float32),
            pltpu.VMEM((NF, CW), jnp.float32),
            pltpu.SemaphoreType.DMA,
            pltpu.SemaphoreType.DMA,
            pltpu.SemaphoreType.DMA,
            pltpu.SemaphoreType.DMA,
        ],
        compiler_params=pltpu.CompilerParams(
            use_tc_tiling_on_sc=True,
            needs_layout_passes=False,
            disable_bounds_checks=True,
        ),
    )
    def k(x_hbm, idx_hbm, out_hbm, idxv, colt, jvt, xpa, xpb,
          outa, outb, isa, isb, osa, osb):
        xps, outs = [xpa, xpb], [outa, outb]
        isems, osems = [isa, isb], [osa, osb]
        wid = lax.axis_index("s") * NC + lax.axis_index("c")
        row0 = wid * rpw
        iota = lax.iota(jnp.int32, L)

        def issue_in(n, b):
            return pltpu.async_copy(
                x_hbm.at[pl.ds((row0 + n * P) * K, PW)], xps[b], isems[b]
            )

        issue_in(0, 0)

        pltpu.sync_copy(idx_hbm, idxv)
        for g in range(NG):
            v = idxv[pl.ds(g * L, L)]
            colt[pl.ds(g * L, L)] = (v >> 7) * 1024 + (v & 127)
            j0 = g * L if g < NG - 1 else NF - L
            jvt[pl.ds(g * L, L)] = iota + j0

        def wait_in(b):
            pltpu.make_async_copy(
                x_hbm.at[pl.ds(0, PW)], xps[b], isems[b]
            ).wait()

        def issue_out(ch, b):
            return pltpu.async_copy(
                outs[b],
                out_hbm.at[:, pl.ds(row0 + ch * CW, CW)],
                osems[b],
            )

        def wait_out(b):
            pltpu.make_async_copy(
                outs[b], out_hbm.at[:, pl.ds(0, CW)], osems[b]
            ).wait()

        def compute_piece(xp, outv, col0):
            @plsc.parallel_loop(0, NG)
            def gbody(g):
                g16 = pl.multiple_of(g * L, L)
                colp = colt[pl.ds(g16, L)]
                jv = jvt[pl.ds(g16, L)]

                @plsc.parallel_loop(0, L)
                def tbody(t):
                    m = (iota + t) & 15
                    ro = ((m >> 3) << 12) + ((m & 7) << 7)
                    gb = colp + ro
                    sb = m + col0
                    for b in range(P // L):
                        vals = plsc.load_gather(xp, [gb + b * 8192])
                        plsc.store_scatter(outv, [jv, sb + b * L], vals)

        def citer(it, _):
            for cc in range(2):
                ch = 2 * it + cc
                for q in range(NPIECE):
                    n = ch * NPIECE + q
                    wait_in(q & 1)

                    @pl.when(n + 1 < TP)
                    def _():
                        issue_in(n + 1, (q + 1) & 1)

                    if q == 0:
                        @pl.when(ch >= 2)
                        def _():
                            wait_out(cc)

                    compute_piece(xps[q & 1], outs[cc], q * P)
                issue_out(ch, cc)
            return 0

        lax.fori_loop(0, C // 2, citer, 0)
        wait_out(0)
        wait_out(1)

    return k


def kernel(x, feature_indices):
    M, K = x.shape
    NF = feature_indices.shape[0]
    G = NF // L
    rem = NF % L
    if rem:
        idx_pad = jnp.concatenate(
            [feature_indices[: G * L], feature_indices[NF - L :]]
        )
    else:
        idx_pad = feature_indices
    NP = idx_pad.shape[0]

    x1 = jnp.transpose(
        x.reshape(M // 8, 8, K // 128, 128), (0, 2, 1, 3)
    ).reshape(-1)

    k = _build(M, K, NF, NP)
    outT = k(x1, idx_pad.astype(jnp.int32))
    return outT.T


# speedup vs baseline: 1.3796x; 1.0283x over previous
import functools

import jax
import jax.numpy as jnp
from jax import lax
from jax.experimental import pallas as pl
from jax.experimental.pallas import tpu as pltpu
from jax.experimental.pallas import tpu_sc as plsc

NC = 2
NS = 16
NW = NC * NS
L = 16


def _build(M, K, NF, NP):
    rpw = M // NW
    CW = 128
    C = rpw // CW
    P = 32
    NPIECE = CW // P
    TP = C * NPIECE
    NG = NP // L
    PW = P * K

    mesh = plsc.VectorSubcoreMesh(core_axis_name="c", subcore_axis_name="s")

    @functools.partial(
        pl.kernel,
        out_type=jax.ShapeDtypeStruct((NF, M), jnp.float32),
        mesh=mesh,
        scratch_types=[
            pltpu.VMEM((NP,), jnp.int32),
            pltpu.VMEM((NP,), jnp.int32),
            pltpu.VMEM((NP,), jnp.int32),
            pltpu.VMEM((PW,), jnp.float32),
            pltpu.VMEM((PW,), jnp.float32),
            pltpu.VMEM((NF, CW), jnp.float32),
            pltpu.VMEM((NF, CW), jnp.float32),
            pltpu.SemaphoreType.DMA,
            pltpu.SemaphoreType.DMA,
            pltpu.SemaphoreType.DMA,
            pltpu.SemaphoreType.DMA,
        ],
        compiler_params=pltpu.CompilerParams(
            use_tc_tiling_on_sc=True,
            needs_layout_passes=False,
            disable_bounds_checks=True,
        ),
    )
    def k(x_hbm, idx_hbm, out_hbm, idxv, colt, jvt, xpa, xpb,
          outa, outb, isa, isb, osa, osb):
        xps, outs = [xpa, xpb], [outa, outb]
        isems, osems = [isa, isb], [osa, osb]
        wid = lax.axis_index("s") * NC + lax.axis_index("c")
        row0 = wid * rpw
        iota = lax.iota(jnp.int32, L)

        def issue_in(n, b):
            return pltpu.async_copy(
                x_hbm.at[pl.ds((row0 + n * P) * K, PW)], xps[b], isems[b]
            )

        issue_in(0, 0)

        pltpu.sync_copy(idx_hbm, idxv)
        for g in range(NG):
            v = idxv[pl.ds(g * L, L)]
            colt[pl.ds(g * L, L)] = (v >> 7) * 1024 + (v & 127)
            j0 = g * L if g < NG - 1 else NF - L
            jvt[pl.ds(g * L, L)] = iota + j0

        def wait_in(b):
            pltpu.make_async_copy(
                x_hbm.at[pl.ds(0, PW)], xps[b], isems[b]
            ).wait()

        def issue_out(ch, b):
            return pltpu.async_copy(
                outs[b],
                out_hbm.at[:, pl.ds(row0 + ch * CW, CW)],
                osems[b],
            )

        def wait_out(b):
            pltpu.make_async_copy(
                outs[b], out_hbm.at[:, pl.ds(0, CW)], osems[b]
            ).wait()

        def compute_piece(xp, outv, col0):
            @plsc.parallel_loop(0, NG)
            def gbody(g):
                g16 = pl.multiple_of(g * L, L)
                colp = colt[pl.ds(g16, L)]
                jv = jvt[pl.ds(g16, L)]

                @plsc.parallel_loop(0, L, unroll=2)
                def tbody(t):
                    m = (iota + t) & 15
                    ro = ((m >> 3) << 12) + ((m & 7) << 7)
                    gb = colp + ro
                    sb = m + col0
                    for b in range(P // L):
                        vals = plsc.load_gather(xp, [gb + b * 8192])
                        plsc.store_scatter(outv, [jv, sb + b * L], vals)

        def citer(it, _):
            for cc in range(2):
                ch = 2 * it + cc
                for q in range(NPIECE):
                    n = ch * NPIECE + q
                    wait_in(q & 1)

                    @pl.when(n + 1 < TP)
                    def _():
                        issue_in(n + 1, (q + 1) & 1)

                    if q == 0:
                        @pl.when(ch >= 2)
                        def _():
                            wait_out(cc)

                    compute_piece(xps[q & 1], outs[cc], q * P)
                issue_out(ch, cc)
            return 0

        lax.fori_loop(0, C // 2, citer, 0)
        wait_out(0)
        wait_out(1)

    return k


def kernel(x, feature_indices):
    M, K = x.shape
    NF = feature_indices.shape[0]
    G = NF // L
    rem = NF % L
    if rem:
        idx_pad = jnp.concatenate(
            [feature_indices[: G * L], feature_indices[NF - L :]]
        )
    else:
        idx_pad = feature_indices
    NP = idx_pad.shape[0]

    x1 = jnp.transpose(
        x.reshape(M // 8, 8, K // 128, 128), (0, 2, 1, 3)
    ).reshape(-1)

    k = _build(M, K, NF, NP)
    outT = k(x1, idx_pad.astype(jnp.int32))
    return outT.T
